# baseline (speedup 1.0000x reference)
import functools

import jax
import jax.numpy as jnp
from jax import lax
from jax.experimental import pallas as pl
from jax.experimental.pallas import tpu as pltpu
from jax.experimental.pallas import tpu_sc as plsc

N_NODES = 10000
D_FEAT = 256
D_LATENT = 128
N_EDGES = 160000

_NC = 2
_NS = 16
_NW = _NC * _NS
_L = 16

_CHUNK = 128
_NCHUNKS = N_EDGES // _CHUNK
_CPW = -(-_NCHUNKS // _NW)


def _encode_matmul(x, W):
    M, K = x.shape
    _, N = W.shape
    BM = 2000

    def body(x_ref, w_ref, z_ref):
        z_ref[...] = jnp.dot(x_ref[...], w_ref[...],
                             preferred_element_type=jnp.float32)

    return pl.pallas_call(
        body,
        grid=(M // BM,),
        in_specs=[
            pl.BlockSpec((BM, K), lambda i: (i, 0)),
            pl.BlockSpec((K, N), lambda i: (0, 0)),
        ],
        out_specs=pl.BlockSpec((BM, N), lambda i: (i, 0)),
        out_shape=jax.ShapeDtypeStruct((M, N), jnp.float32),
    )(x, W)


def _decode_body(z_hbm, src_hbm, dst_hbm, out_hbm,
                 idx_s, idx_d, rows_s, rows_d, out_v, sem_s, sem_d):
    wid = lax.axis_index("s") * _NC + lax.axis_index("c")

    def chunk_body(i, carry):
        c = wid + i * _NW

        @pl.when(c < _NCHUNKS)
        def _():
            base = c * _CHUNK
            pltpu.sync_copy(src_hbm.at[pl.ds(base, _CHUNK)], idx_s)
            pltpu.sync_copy(dst_hbm.at[pl.ds(base, _CHUNK)], idx_d)
            cp_s = pltpu.async_copy(z_hbm.at[idx_s], rows_s, sem_s)
            cp_d = pltpu.async_copy(z_hbm.at[idx_d], rows_d, sem_d)
            cp_s.wait()
            cp_d.wait()
            def group_body(g, carry):
                rowv = lax.iota(jnp.int32, _L) + g * _L
                accs = [jnp.zeros((_L,), jnp.float32) for _ in range(4)]
                for j in range(D_LATENT):
                    jv = jnp.full((_L,), j, jnp.int32)
                    sv = plsc.load_gather(rows_s, [rowv, jv])
                    dv = plsc.load_gather(rows_d, [rowv, jv])
                    accs[j % 4] = accs[j % 4] + sv * dv
                acc = (accs[0] + accs[1]) + (accs[2] + accs[3])
                e = jnp.exp(-jnp.abs(acc))
                sig = jnp.where(acc >= 0.0, 1.0 / (1.0 + e), e / (1.0 + e))
                out_v[pl.ds(g * _L, _L)] = sig
                return carry

            lax.fori_loop(0, _CHUNK // _L, group_body, 0)
            pltpu.sync_copy(out_v, out_hbm.at[pl.ds(base, _CHUNK)])

        return carry

    lax.fori_loop(0, _CPW, chunk_body, 0)


def _decode(z, src, dst):
    mesh = plsc.VectorSubcoreMesh(core_axis_name="c", subcore_axis_name="s")
    k = functools.partial(
        pl.kernel,
        out_type=jax.ShapeDtypeStruct((N_EDGES,), jnp.float32),
        mesh=mesh,
        scratch_types=[
            pltpu.VMEM((_CHUNK,), jnp.int32),
            pltpu.VMEM((_CHUNK,), jnp.int32),
            pltpu.VMEM((_CHUNK, D_LATENT), jnp.float32),
            pltpu.VMEM((_CHUNK, D_LATENT), jnp.float32),
            pltpu.VMEM((_CHUNK,), jnp.float32),
            pltpu.SemaphoreType.DMA,
            pltpu.SemaphoreType.DMA,
        ],
        compiler_params=pltpu.CompilerParams(needs_layout_passes=False),
    )(_decode_body)
    return k(z, src, dst)


def kernel(x, edge_index, W):
    z = _encode_matmul(x, W)
    ei = edge_index.astype(jnp.int32)
    return _decode(z, ei[0], ei[1])


# speedup vs baseline: 3.5767x; 3.5767x over previous
import functools

import jax
import jax.numpy as jnp
from jax import lax
from jax.experimental import pallas as pl
from jax.experimental.pallas import tpu as pltpu
from jax.experimental.pallas import tpu_sc as plsc

N_NODES = 10000
D_FEAT = 256
D_LATENT = 128
N_EDGES = 160000

_NC = 2
_NS = 16
_NW = _NC * _NS
_L = 16

_CHUNK = 128
_NCHUNKS = N_EDGES // _CHUNK
_CPW = -(-_NCHUNKS // _NW)


def _encode_matmul(x, W):
    M, K = x.shape
    _, N = W.shape
    BM = 2000

    def body(x_ref, w_ref, z_ref):
        z_ref[...] = jnp.dot(x_ref[...], w_ref[...],
                             preferred_element_type=jnp.float32)

    return pl.pallas_call(
        body,
        grid=(M // BM,),
        in_specs=[
            pl.BlockSpec((BM, K), lambda i: (i, 0)),
            pl.BlockSpec((K, N), lambda i: (0, 0)),
        ],
        out_specs=pl.BlockSpec((BM, N), lambda i: (i, 0)),
        out_shape=jax.ShapeDtypeStruct((M, N), jnp.float32),
    )(x, W)


def _decode_body(z_hbm, src_hbm, dst_hbm, out_hbm,
                 idx_s, idx_d, rows_s, rows_d, tbuf, out_v, sem_s, sem_d):
    wid = lax.axis_index("s") * _NC + lax.axis_index("c")

    def chunk_body(i, carry):
        c = wid + i * _NW

        @pl.when(c < _NCHUNKS)
        def _():
            base = c * _CHUNK
            pltpu.sync_copy(src_hbm.at[pl.ds(base, _CHUNK)], idx_s)
            pltpu.sync_copy(dst_hbm.at[pl.ds(base, _CHUNK)], idx_d)
            cp_s = pltpu.async_copy(z_hbm.at[idx_s], rows_s, sem_s)
            cp_d = pltpu.async_copy(z_hbm.at[idx_d], rows_d, sem_d)
            cp_s.wait()
            cp_d.wait()
            def group_body(g, carry):
                for e in range(_L):
                    row = g * _L + e
                    accs = [
                        rows_s[row, pl.ds(4 * k * _L, _L)]
                        * rows_d[row, pl.ds(4 * k * _L, _L)]
                        for k in range(2)
                    ]
                    for k in range(D_LATENT // _L):
                        if k % 4 != 0:
                            kk = (k % 4) // 2
                            accs[kk] = accs[kk] + (
                                rows_s[row, pl.ds(k * _L, _L)]
                                * rows_d[row, pl.ds(k * _L, _L)]
                            )
                    tbuf[pl.ds(e * 17, _L)] = accs[0] + accs[1]
                rowv = lax.iota(jnp.int32, _L) * 17
                accs = [
                    plsc.load_gather(tbuf, [rowv]),
                    plsc.load_gather(tbuf, [rowv + 1]),
                ]
                for k in range(2, _L):
                    accs[k % 2] = accs[k % 2] + plsc.load_gather(
                        tbuf, [rowv + k])
                acc = accs[0] + accs[1]
                e = jnp.exp(-jnp.abs(acc))
                sig = jnp.where(acc >= 0.0, 1.0 / (1.0 + e), e / (1.0 + e))
                out_v[pl.ds(g * _L, _L)] = sig
                return carry

            lax.fori_loop(0, _CHUNK // _L, group_body, 0)
            pltpu.sync_copy(out_v, out_hbm.at[pl.ds(base, _CHUNK)])

        return carry

    lax.fori_loop(0, _CPW, chunk_body, 0)


def _decode(z, src, dst):
    mesh = plsc.VectorSubcoreMesh(core_axis_name="c", subcore_axis_name="s")
    k = functools.partial(
        pl.kernel,
        out_type=jax.ShapeDtypeStruct((N_EDGES,), jnp.float32),
        mesh=mesh,
        scratch_types=[
            pltpu.VMEM((_CHUNK,), jnp.int32),
            pltpu.VMEM((_CHUNK,), jnp.int32),
            pltpu.VMEM((_CHUNK, D_LATENT), jnp.float32),
            pltpu.VMEM((_CHUNK, D_LATENT), jnp.float32),
            pltpu.VMEM((_L * 17,), jnp.float32),
            pltpu.VMEM((_CHUNK,), jnp.float32),
            pltpu.SemaphoreType.DMA,
            pltpu.SemaphoreType.DMA,
        ],
        compiler_params=pltpu.CompilerParams(needs_layout_passes=False),
    )(_decode_body)
    return k(z, src, dst)


def kernel(x, edge_index, W):
    z = _encode_matmul(x, W)
    ei = edge_index.astype(jnp.int32)
    return _decode(z, ei[0], ei[1])


# speedup vs baseline: 6.6363x; 1.8554x over previous
import functools

import jax
import jax.numpy as jnp
from jax import lax
from jax.experimental import pallas as pl
from jax.experimental.pallas import tpu as pltpu
from jax.experimental.pallas import tpu_sc as plsc

N_NODES = 10000
D_FEAT = 256
D_LATENT = 128
N_EDGES = 160000

_NC = 2
_NS = 16
_NW = _NC * _NS
_L = 16

_EPW = N_EDGES // _NW
_CHUNK = 128
_NCH = -(-_EPW // _CHUNK)
_LAST = _EPW - _CHUNK
_NPAIR = _NCH // 2


def _encode_matmul(x, W):
    M, K = x.shape
    _, N = W.shape
    BM = 2000

    def body(x_ref, w_ref, z_ref):
        z_ref[...] = jnp.dot(x_ref[...], w_ref[...],
                             preferred_element_type=jnp.float32)

    return pl.pallas_call(
        body,
        grid=(M // BM,),
        in_specs=[
            pl.BlockSpec((BM, K), lambda i: (i, 0)),
            pl.BlockSpec((K, N), lambda i: (0, 0)),
        ],
        out_specs=pl.BlockSpec((BM, N), lambda i: (i, 0)),
        out_shape=jax.ShapeDtypeStruct((M, N), jnp.float32),
    )(x, W)


def _chunk_base(c):
    return jnp.minimum(c * _CHUNK, _LAST)


def _decode_body(z_hbm, src_hbm, dst_hbm, out_hbm,
                 idx_s, idx_d, rows_s0, rows_d0, rows_s1, rows_d1,
                 tbuf, out_v,
                 sem_s0, sem_d0, sem_s1, sem_d1):
    wid = lax.axis_index("s") * _NC + lax.axis_index("c")
    ebase = wid * _EPW

    pltpu.sync_copy(src_hbm.at[pl.ds(ebase, _EPW)], idx_s)
    pltpu.sync_copy(dst_hbm.at[pl.ds(ebase, _EPW)], idx_d)

    def issue(c, rs, rd, ss, sd):
        b = _chunk_base(c)
        pltpu.async_copy(z_hbm.at[idx_s.at[pl.ds(b, _CHUNK)]], rs, ss)
        pltpu.async_copy(z_hbm.at[idx_d.at[pl.ds(b, _CHUNK)]], rd, sd)

    def wait(rs, rd, ss, sd):
        pltpu.make_async_copy(z_hbm.at[idx_s.at[pl.ds(0, _CHUNK)]],
                              rs, ss).wait()
        pltpu.make_async_copy(z_hbm.at[idx_d.at[pl.ds(0, _CHUNK)]],
                              rd, sd).wait()

    def compute(c, rows_s, rows_d):
        b = _chunk_base(c)

        def group_body(g, carry):
            for e in range(_L):
                row = g * _L + e
                accs = [
                    rows_s[row, pl.ds(4 * k * _L, _L)]
                    * rows_d[row, pl.ds(4 * k * _L, _L)]
                    for k in range(2)
                ]
                for k in range(D_LATENT // _L):
                    if k % 4 != 0:
                        kk = (k % 4) // 2
                        accs[kk] = accs[kk] + (
                            rows_s[row, pl.ds(k * _L, _L)]
                            * rows_d[row, pl.ds(k * _L, _L)]
                        )
                tbuf[pl.ds(e * 17, _L)] = accs[0] + accs[1]
            rowv = lax.iota(jnp.int32, _L) * 17
            accs = [
                plsc.load_gather(tbuf, [rowv]),
                plsc.load_gather(tbuf, [rowv + 1]),
            ]
            for k in range(2, _L):
                accs[k % 2] = accs[k % 2] + plsc.load_gather(tbuf, [rowv + k])
            acc = accs[0] + accs[1]
            ex = jnp.exp(-jnp.abs(acc))
            sig = jnp.where(acc >= 0.0, 1.0 / (1.0 + ex), ex / (1.0 + ex))
            out_v[pl.ds(b + g * _L, _L)] = sig
            return carry

        lax.fori_loop(0, _CHUNK // _L, group_body, 0)

    issue(0, rows_s0, rows_d0, sem_s0, sem_d0)

    def pair_body(p, carry):
        c0 = 2 * p
        issue(c0 + 1, rows_s1, rows_d1, sem_s1, sem_d1)
        wait(rows_s0, rows_d0, sem_s0, sem_d0)
        compute(c0, rows_s0, rows_d0)

        @pl.when(p < _NPAIR - 1)
        def _():
            issue(c0 + 2, rows_s0, rows_d0, sem_s0, sem_d0)

        wait(rows_s1, rows_d1, sem_s1, sem_d1)
        compute(c0 + 1, rows_s1, rows_d1)
        return carry

    lax.fori_loop(0, _NPAIR, pair_body, 0)
    pltpu.sync_copy(out_v, out_hbm.at[pl.ds(ebase, _EPW)])


def _decode(z, src, dst):
    mesh = plsc.VectorSubcoreMesh(core_axis_name="c", subcore_axis_name="s")
    k = functools.partial(
        pl.kernel,
        out_type=jax.ShapeDtypeStruct((N_EDGES,), jnp.float32),
        mesh=mesh,
        scratch_types=[
            pltpu.VMEM((_EPW,), jnp.int32),
            pltpu.VMEM((_EPW,), jnp.int32),
            pltpu.VMEM((_CHUNK, D_LATENT), jnp.float32),
            pltpu.VMEM((_CHUNK, D_LATENT), jnp.float32),
            pltpu.VMEM((_CHUNK, D_LATENT), jnp.float32),
            pltpu.VMEM((_CHUNK, D_LATENT), jnp.float32),
            pltpu.VMEM((_L * 17,), jnp.float32),
            pltpu.VMEM((_EPW,), jnp.float32),
            pltpu.SemaphoreType.DMA,
            pltpu.SemaphoreType.DMA,
            pltpu.SemaphoreType.DMA,
            pltpu.SemaphoreType.DMA,
        ],
        compiler_params=pltpu.CompilerParams(needs_layout_passes=False),
    )(_decode_body)
    return k(z, src, dst)


def kernel(x, edge_index, W):
    z = _encode_matmul(x, W)
    ei = edge_index.astype(jnp.int32)
    return _decode(z, ei[0], ei[1])
